# baseline (speedup 1.0000x reference)
import functools

import jax
import jax.numpy as jnp
from jax import lax
from jax.experimental import pallas as pl
from jax.experimental.pallas import tpu as pltpu
from jax.experimental.pallas import tpu_sc as plsc

_CHUNK = 16
_NBUF = 4


@functools.cache
def _sc_gather_call(n_rows: int, n_cols: int, chunk: int, nbuf: int):
    info = plsc.get_sparse_core_info()
    nw = info.num_cores * info.num_subcores
    rows_per_worker = n_rows // nw
    n_chunks = rows_per_worker // chunk
    mesh = plsc.VectorSubcoreMesh(core_axis_name="c", subcore_axis_name="s")

    @functools.partial(
        pl.kernel,
        mesh=mesh,
        out_type=(
            jax.ShapeDtypeStruct((n_rows, n_cols), jnp.float32),
            jax.ShapeDtypeStruct((n_rows,), jnp.int32),
        ),
        scratch_types=[
            pltpu.VMEM((rows_per_worker,), jnp.int32),
            pltpu.VMEM((rows_per_worker,), jnp.int32),
            pltpu.VMEM_SHARED(
                (info.num_subcores, nbuf, chunk, n_cols), jnp.float32
            ),
        ]
        + [pltpu.SemaphoreType.DMA] * (2 * nbuf + 1),
    )
    def gather(x_hbm, idx_hbm, perm_hbm, out_hbm, perm_out_hbm, idx_v, perm_v,
               rows_v, *sems):
        gsem = sems[:nbuf]
        ssem = sems[nbuf : 2 * nbuf]
        psem = sems[2 * nbuf]
        sid = lax.axis_index("s")
        wid = sid * info.num_cores + lax.axis_index("c")
        base = wid * rows_per_worker
        pltpu.sync_copy(idx_hbm.at[pl.ds(base, rows_per_worker)], idx_v)

        def start_gather(c, b):
            vec = idx_v[pl.ds(c * chunk, chunk)]
            for k in range(chunk):
                pltpu.async_copy(
                    x_hbm.at[pl.ds(vec[k], 1)],
                    rows_v.at[sid, b, pl.ds(k, 1)],
                    gsem[b],
                )

        def wait_gather(c, b):
            pltpu.make_async_copy(
                x_hbm.at[pl.ds(0, chunk)], rows_v.at[sid, b], gsem[b]
            ).wait()

        def start_scatter(c, b):
            pltpu.async_copy(
                rows_v.at[sid, b],
                out_hbm.at[pl.ds(base + c * chunk, chunk)],
                ssem[b],
            )

        def wait_scatter(c, b):
            pltpu.make_async_copy(
                rows_v.at[sid, b],
                out_hbm.at[pl.ds(base + c * chunk, chunk)],
                ssem[b],
            ).wait()

        for b in range(nbuf):
            start_gather(b, b)

        pltpu.sync_copy(perm_hbm.at[pl.ds(base, rows_per_worker)], perm_v)
        pltpu.async_copy(
            perm_v, perm_out_hbm.at[pl.ds(base, rows_per_worker)], psem
        )

        def body(p, carry):
            for j in range(nbuf):
                c = p * nbuf + j
                wait_gather(c, j)
                start_scatter(c, j)

                @pl.when(jnp.logical_and(c >= 2, c <= n_chunks - nbuf + 1))
                def _():
                    b1 = (j - 2) % nbuf
                    wait_scatter(c - 2, b1)
                    start_gather(c + nbuf - 2, b1)

            return carry

        lax.fori_loop(0, n_chunks // nbuf, body, 0)
        for i in range(nbuf):
            c = n_chunks - nbuf + i
            wait_scatter(c, c % nbuf)
        pltpu.make_async_copy(
            perm_v, perm_out_hbm.at[pl.ds(base, rows_per_worker)], psem
        ).wait()

    return gather


@functools.cache
def _perm_constants(B: int, L: int):
    import numpy as np

    with jax.ensure_compile_time_eval():
        perm1d = np.asarray(jax.random.permutation(jax.random.key(42), L))
    perm = np.tile(perm1d[None, :], (B, 1))
    src = (
        np.arange(B, dtype=np.int32)[:, None] * L + perm1d[None, :]
    ).reshape(-1)
    return jnp.asarray(perm.reshape(-1).astype(np.int32)), jnp.asarray(
        src.astype(np.int32)
    )


def kernel(x):
    B, L, C = x.shape
    perm_flat, src = _perm_constants(B, L)
    out, perm_out = _sc_gather_call(B * L, C, _CHUNK, _NBUF)(
        x.reshape(B * L, C), src, perm_flat
    )
    return out.reshape(B, L, C), perm_out.reshape(B, L)


# speedup vs baseline: 1.0105x; 1.0105x over previous
import functools

import jax
import jax.numpy as jnp
from jax import lax
from jax.experimental import pallas as pl
from jax.experimental.pallas import tpu as pltpu
from jax.experimental.pallas import tpu_sc as plsc

_CHUNK = 16
_NBUF = 4


@functools.cache
def _sc_perm_call(B: int, L: int, C: int, chunk: int, nbuf: int):
    info = plsc.get_sparse_core_info()
    nw = info.num_cores * info.num_subcores
    rows_per_worker = B * L // nw
    per_batch = nw // B
    n_chunks = rows_per_worker // chunk
    mesh = plsc.VectorSubcoreMesh(core_axis_name="c", subcore_axis_name="s")

    @functools.partial(
        pl.kernel,
        mesh=mesh,
        out_type=(
            jax.ShapeDtypeStruct((B, L, C), jnp.float32),
            jax.ShapeDtypeStruct((B, L), jnp.int32),
        ),
        scratch_types=[
            pltpu.VMEM((rows_per_worker,), jnp.int32),
            pltpu.VMEM_SHARED(
                (info.num_subcores, nbuf, chunk, C), jnp.float32
            ),
        ]
        + [pltpu.SemaphoreType.DMA] * (2 * nbuf + 1),
    )
    def gather(x_hbm, perm_hbm, out_hbm, perm_out_hbm, idx_v, rows_v, *sems):
        gsem = sems[:nbuf]
        ssem = sems[nbuf : 2 * nbuf]
        psem = sems[2 * nbuf]
        sid = lax.axis_index("s")
        wid = sid * info.num_cores + lax.axis_index("c")
        bw = wid // per_batch
        off = (wid % per_batch) * rows_per_worker
        pltpu.sync_copy(
            perm_hbm.at[bw, pl.ds(off, rows_per_worker)], idx_v
        )

        def start_gather(c, b):
            vec = idx_v[pl.ds(c * chunk, chunk)]
            for k in range(chunk):
                pltpu.async_copy(
                    x_hbm.at[bw, pl.ds(vec[k], 1)],
                    rows_v.at[sid, b, pl.ds(k, 1)],
                    gsem[b],
                )

        def wait_gather(c, b):
            pltpu.make_async_copy(
                x_hbm.at[bw, pl.ds(0, chunk)], rows_v.at[sid, b], gsem[b]
            ).wait()

        def start_scatter(c, b):
            pltpu.async_copy(
                rows_v.at[sid, b],
                out_hbm.at[bw, pl.ds(off + c * chunk, chunk)],
                ssem[b],
            )

        def wait_scatter(c, b):
            pltpu.make_async_copy(
                rows_v.at[sid, b],
                out_hbm.at[bw, pl.ds(off + c * chunk, chunk)],
                ssem[b],
            ).wait()

        for b in range(nbuf):
            start_gather(b, b)

        pltpu.async_copy(
            idx_v, perm_out_hbm.at[bw, pl.ds(off, rows_per_worker)], psem
        )

        def body(p, carry):
            for j in range(nbuf):
                c = p * nbuf + j
                wait_gather(c, j)
                start_scatter(c, j)

                @pl.when(jnp.logical_and(c >= 2, c <= n_chunks - nbuf + 1))
                def _():
                    b1 = (j - 2) % nbuf
                    wait_scatter(c - 2, b1)
                    start_gather(c + nbuf - 2, b1)

            return carry

        lax.fori_loop(0, n_chunks // nbuf, body, 0)
        for i in range(nbuf):
            c = n_chunks - nbuf + i
            wait_scatter(c, c % nbuf)
        pltpu.make_async_copy(
            idx_v, perm_out_hbm.at[bw, pl.ds(off, rows_per_worker)], psem
        ).wait()

    return gather


@functools.cache
def _perm_constant(B: int, L: int):
    import numpy as np

    with jax.ensure_compile_time_eval():
        perm1d = np.asarray(jax.random.permutation(jax.random.key(42), L))
    return jnp.asarray(np.tile(perm1d[None, :], (B, 1)).astype(np.int32))


def kernel(x):
    B, L, C = x.shape
    perm = _perm_constant(B, L)
    return _sc_perm_call(B, L, C, _CHUNK, _NBUF)(x, perm)
